# baseline (speedup 1.0000x reference)
import functools

import jax
import jax.numpy as jnp
from jax.experimental import pallas as pl

_N = 50000
_E = 800000
_G = 512
_TILE = 1000


def _full(spec_shape):
    nd = len(spec_shape)
    return pl.BlockSpec(spec_shape, lambda i: (0,) * nd)


def _rows(cols):
    return pl.BlockSpec((_TILE, cols), lambda i: (i, 0))


def _elu(v):
    return jnp.where(v > 0, v, jnp.exp(jnp.minimum(v, 0.0)) - 1.0)


def _bn_elu(v, g, b, rm, rv):
    return _elu(g * (v - rm) * jax.lax.rsqrt(rv + 1e-5) + b)


def _k1_body(x_ref, w_ref, a_ref, h_ref, s_ref):
    h = jnp.dot(x_ref[...], w_ref[...], preferred_element_type=jnp.float32)
    h_ref[...] = h
    s_ref[...] = jnp.dot(h, a_ref[...], preferred_element_type=jnp.float32)


def _k2_body(agg_ref, b1_ref, g_ref, bb_ref, rm_ref, rv_ref, wg_ref,
             y_ref, hg_ref):
    y = _bn_elu(agg_ref[...] + b1_ref[...], g_ref[...], bb_ref[...],
                rm_ref[...], rv_ref[...])
    y_ref[...] = y
    hg_ref[...] = jnp.dot(y, wg_ref[...], preferred_element_type=jnp.float32)


def _k3_body(agg_ref, bg_ref, g_ref, bb_ref, rm_ref, rv_ref, wsr_ref,
             x3_ref, xr_ref):
    x3 = _bn_elu(agg_ref[...] + bg_ref[...], g_ref[...], bb_ref[...],
                 rm_ref[...], rv_ref[...])
    x3_ref[...] = x3
    xr_ref[...] = jnp.dot(x3, wsr_ref[...], preferred_element_type=jnp.float32)


def _k4_body(aggs_ref, wsl_ref, bsl_ref, xr_ref, g_ref, bb_ref, rm_ref,
             rv_ref, w2_ref, a2_ref, h2_ref, s2_ref):
    s = jnp.dot(aggs_ref[...], wsl_ref[...],
                preferred_element_type=jnp.float32) + bsl_ref[...] + xr_ref[...]
    x4 = _bn_elu(s, g_ref[...], bb_ref[...], rm_ref[...], rv_ref[...])
    h2 = jnp.dot(x4, w2_ref[...], preferred_element_type=jnp.float32)
    h2_ref[...] = h2
    s2_ref[...] = jnp.dot(h2, a2_ref[...], preferred_element_type=jnp.float32)


def _k5_body(agg_ref, b2_ref, g_ref, bb_ref, rm_ref, rv_ref, x5_ref):
    x5_ref[...] = _bn_elu(agg_ref[...] + b2_ref[...], g_ref[...], bb_ref[...],
                          rm_ref[...], rv_ref[...])


def _k6_body(z_ref, w1, b1, g1, bb1, rm1, rv1, w2, b2, g2, bb2, rm2, rv2,
             w3, b3, w4, b4, w5, b5, out_ref):
    z = _elu(jnp.dot(z_ref[...], w1[...],
                     preferred_element_type=jnp.float32) + b1[...])
    z = g1[...] * (z - rm1[...]) * jax.lax.rsqrt(rv1[...] + 1e-5) + bb1[...]
    z = _elu(jnp.dot(z, w2[...],
                     preferred_element_type=jnp.float32) + b2[...])
    z = g2[...] * (z - rm2[...]) * jax.lax.rsqrt(rv2[...] + 1e-5) + bb2[...]
    z = _elu(jnp.dot(z, w3[...],
                     preferred_element_type=jnp.float32) + b3[...])
    z = _elu(jnp.dot(z, w4[...],
                     preferred_element_type=jnp.float32) + b4[...])
    out_ref[...] = jnp.dot(z, w5[...],
                           preferred_element_type=jnp.float32) + b5[...]


def _attn_mat(a_s, a_d):
    heads, ch = a_s.shape
    eye = jnp.eye(heads, dtype=a_s.dtype)
    left = (a_s[:, :, None] * eye[:, None, :]).reshape(heads * ch, heads)
    right = (a_d[:, :, None] * eye[:, None, :]).reshape(heads * ch, heads)
    return jnp.concatenate([left, right], axis=1)


def _r2(v):
    return v.reshape(1, -1)


def _seg_sum(d, s, n):
    return jax.ops.segment_sum(d, s, num_segments=n)


def _gat_edges(h, ssd, src, dst, heads, ch):
    n = h.shape[0]
    alpha = jax.nn.leaky_relu(ssd[src, :heads] + ssd[dst, heads:], 0.2)
    m = jax.ops.segment_max(alpha, dst, num_segments=n)
    m = jnp.where(jnp.isfinite(m), m, 0.0)
    e = jnp.exp(alpha - m[dst])
    attn = e / (_seg_sum(e, dst, n)[dst] + 1e-16)
    hh = h.reshape(n, heads, ch)
    return _seg_sum(hh[src] * attn[:, :, None], dst, n)


@jax.jit
def kernel(x, params, edge_index, batch):
    p = params
    f32 = jnp.float32
    grid = (_N // _TILE,)

    loop = jnp.arange(_N, dtype=edge_index.dtype)
    src = jnp.concatenate([edge_index[0], loop])
    dst = jnp.concatenate([edge_index[1], loop])

    a1 = _attn_mat(p['as1'], p['ad1'])
    h1, s1 = pl.pallas_call(
        _k1_body,
        grid=grid,
        in_specs=[_rows(28), _full((28, 256)), _full((256, 8))],
        out_specs=[_rows(256), _rows(8)],
        out_shape=[jax.ShapeDtypeStruct((_N, 256), f32),
                   jax.ShapeDtypeStruct((_N, 8), f32)],
    )(x, p['W1'], a1)

    agg1 = _gat_edges(h1, s1, src, dst, 4, 64).reshape(_N, 256)

    g1, b1, rm1, rv1 = p['bn1']
    y, hg = pl.pallas_call(
        _k2_body,
        grid=grid,
        in_specs=[_rows(256)] + [_full((1, 256))] * 5 + [_full((256, 64))],
        out_specs=[_rows(256), _rows(64)],
        out_shape=[jax.ShapeDtypeStruct((_N, 256), f32),
                   jax.ShapeDtypeStruct((_N, 64), f32)],
    )(agg1, _r2(p['b1']), _r2(g1), _r2(b1), _r2(rm1), _r2(rv1), p['Wg'])
    del y

    deg = _seg_sum(jnp.ones(src.shape[0], f32), dst, _N)
    dinv = jnp.where(deg > 0, deg ** -0.5, 0.0)
    aggg = _seg_sum(hg[src] * (dinv[src] * dinv[dst])[:, None], dst, _N)

    g2, b2, rm2, rv2 = p['bn2']
    x3, xr = pl.pallas_call(
        _k3_body,
        grid=grid,
        in_specs=[_rows(64)] + [_full((1, 64))] * 5 + [_full((64, 64))],
        out_specs=[_rows(64), _rows(64)],
        out_shape=[jax.ShapeDtypeStruct((_N, 64), f32),
                   jax.ShapeDtypeStruct((_N, 64), f32)],
    )(aggg, _r2(p['bg']), _r2(g2), _r2(b2), _r2(rm2), _r2(rv2), p['Wsr'])

    es, ed = edge_index[0], edge_index[1]
    cnt = _seg_sum(jnp.ones(es.shape[0], f32), ed, _N)
    aggs = _seg_sum(x3[es], ed, _N) / jnp.maximum(cnt, 1.0)[:, None]

    g3, b3, rm3, rv3 = p['bn3']
    a2 = _attn_mat(p['as2'], p['ad2'])
    h2, s2 = pl.pallas_call(
        _k4_body,
        grid=grid,
        in_specs=[_rows(64), _full((64, 64)), _full((1, 64)), _rows(64)]
                 + [_full((1, 64))] * 4 + [_full((64, 256)), _full((256, 8))],
        out_specs=[_rows(256), _rows(8)],
        out_shape=[jax.ShapeDtypeStruct((_N, 256), f32),
                   jax.ShapeDtypeStruct((_N, 8), f32)],
    )(aggs, p['Wsl'], _r2(p['bsl']), xr, _r2(g3), _r2(b3), _r2(rm3),
      _r2(rv3), p['W2'], a2)

    agg2 = _gat_edges(h2, s2, src, dst, 4, 64).mean(1)

    g4, b4, rm4, rv4 = p['bn4']
    x5 = pl.pallas_call(
        _k5_body,
        grid=grid,
        in_specs=[_rows(64)] + [_full((1, 64))] * 5,
        out_specs=_rows(64),
        out_shape=jax.ShapeDtypeStruct((_N, 64), f32),
    )(agg2, _r2(p['b2']), _r2(g4), _r2(b4), _r2(rm4), _r2(rv4))

    gcnt = _seg_sum(jnp.ones(_N, f32), batch, _G)
    xm = _seg_sum(x5, batch, _G) / jnp.maximum(gcnt, 1.0)[:, None]
    xx = jax.ops.segment_max(x5, batch, num_segments=_G)
    xx = jnp.where(jnp.isfinite(xx), xx, 0.0)
    z0 = jnp.concatenate([xm, xx], axis=1)

    gm1, bm1, rmm1, rvm1 = p['bnm1']
    gm2, bm2, rmm2, rvm2 = p['bnm2']
    out = pl.pallas_call(
        _k6_body,
        grid=(1,),
        in_specs=[_full((_G, 128)),
                  _full((128, 64)), _full((1, 64)), _full((1, 64)),
                  _full((1, 64)), _full((1, 64)), _full((1, 64)),
                  _full((64, 32)), _full((1, 32)), _full((1, 32)),
                  _full((1, 32)), _full((1, 32)), _full((1, 32)),
                  _full((32, 16)), _full((1, 16)),
                  _full((16, 32)), _full((1, 32)),
                  _full((32, 1)), _full((1, 1))],
        out_specs=_full((_G, 1)),
        out_shape=jax.ShapeDtypeStruct((_G, 1), f32),
    )(z0, p['L1W'], _r2(p['L1b']), _r2(gm1), _r2(bm1), _r2(rmm1), _r2(rvm1),
      p['L2W'], _r2(p['L2b']), _r2(gm2), _r2(bm2), _r2(rmm2), _r2(rvm2),
      p['L3W'], _r2(p['L3b']), p['L4W'], _r2(p['L4b']),
      p['L5W'], _r2(p['L5b']))
    return out


# speedup vs baseline: 2.0057x; 2.0057x over previous
import functools

import jax
import jax.numpy as jnp
from jax.experimental import pallas as pl

_N = 50000
_E = 800000
_G = 512
_TILE = 1000


def _full(spec_shape):
    nd = len(spec_shape)
    return pl.BlockSpec(spec_shape, lambda i: (0,) * nd)


def _rows(cols):
    return pl.BlockSpec((_TILE, cols), lambda i: (i, 0))


def _elu(v):
    return jnp.where(v > 0, v, jnp.exp(jnp.minimum(v, 0.0)) - 1.0)


def _bn_elu(v, g, b, rm, rv):
    return _elu(g * (v - rm) * jax.lax.rsqrt(rv + 1e-5) + b)


def _k1_body(x_ref, w_ref, a_ref, h_ref, s_ref):
    h = jnp.dot(x_ref[...], w_ref[...], preferred_element_type=jnp.float32)
    h_ref[...] = h
    s_ref[...] = jnp.dot(h, a_ref[...], preferred_element_type=jnp.float32)


def _k2_body(agg_ref, b1_ref, g_ref, bb_ref, rm_ref, rv_ref, wg_ref,
             hg_ref):
    y = _bn_elu(agg_ref[...] + b1_ref[...], g_ref[...], bb_ref[...],
                rm_ref[...], rv_ref[...])
    hg_ref[...] = jnp.dot(y, wg_ref[...], preferred_element_type=jnp.float32)


def _k3_body(agg_ref, bg_ref, g_ref, bb_ref, rm_ref, rv_ref, wsr_ref,
             x3_ref, xr_ref):
    x3 = _bn_elu(agg_ref[...] + bg_ref[...], g_ref[...], bb_ref[...],
                 rm_ref[...], rv_ref[...])
    x3_ref[...] = x3
    xr_ref[...] = jnp.dot(x3, wsr_ref[...], preferred_element_type=jnp.float32)


def _k4_body(aggs_ref, wsl_ref, bsl_ref, xr_ref, g_ref, bb_ref, rm_ref,
             rv_ref, w2_ref, a2_ref, h2_ref, s2_ref):
    s = jnp.dot(aggs_ref[...], wsl_ref[...],
                preferred_element_type=jnp.float32) + bsl_ref[...] + xr_ref[...]
    x4 = _bn_elu(s, g_ref[...], bb_ref[...], rm_ref[...], rv_ref[...])
    h2 = jnp.dot(x4, w2_ref[...], preferred_element_type=jnp.float32)
    h2_ref[...] = h2
    s2_ref[...] = jnp.dot(h2, a2_ref[...], preferred_element_type=jnp.float32)


def _k5_body(agg_ref, b2_ref, g_ref, bb_ref, rm_ref, rv_ref, x5_ref):
    x5_ref[...] = _bn_elu(agg_ref[...] + b2_ref[...], g_ref[...], bb_ref[...],
                          rm_ref[...], rv_ref[...])


def _k6_body(z_ref, w1, b1, g1, bb1, rm1, rv1, w2, b2, g2, bb2, rm2, rv2,
             w3, b3, w4, b4, w5, b5, out_ref):
    z = _elu(jnp.dot(z_ref[...], w1[...],
                     preferred_element_type=jnp.float32) + b1[...])
    z = g1[...] * (z - rm1[...]) * jax.lax.rsqrt(rv1[...] + 1e-5) + bb1[...]
    z = _elu(jnp.dot(z, w2[...],
                     preferred_element_type=jnp.float32) + b2[...])
    z = g2[...] * (z - rm2[...]) * jax.lax.rsqrt(rv2[...] + 1e-5) + bb2[...]
    z = _elu(jnp.dot(z, w3[...],
                     preferred_element_type=jnp.float32) + b3[...])
    z = _elu(jnp.dot(z, w4[...],
                     preferred_element_type=jnp.float32) + b4[...])
    out_ref[...] = jnp.dot(z, w5[...],
                           preferred_element_type=jnp.float32) + b5[...]


def _attn_mat(a_s, a_d):
    heads, ch = a_s.shape
    eye = jnp.eye(heads, dtype=a_s.dtype)
    left = (a_s[:, :, None] * eye[:, None, :]).reshape(heads * ch, heads)
    right = (a_d[:, :, None] * eye[:, None, :]).reshape(heads * ch, heads)
    return jnp.concatenate([left, right], axis=1)


def _r2(v):
    return v.reshape(1, -1)


def _seg_sum(d, s, n):
    return jax.ops.segment_sum(d, s, num_segments=n)


def _gat_edges(h, ssd, src, dst, heads, ch):
    n = h.shape[0]
    s_src = ssd[:, :heads]
    s_dst = ssd[:, heads:]
    alpha = jax.nn.leaky_relu(s_src[src] + s_dst[dst], 0.2)
    m = jax.ops.segment_max(alpha, dst, num_segments=n)
    m = jnp.where(jnp.isfinite(m), m, 0.0)
    e = jnp.exp(alpha - m[dst])
    attn = e / (_seg_sum(e, dst, n)[dst] + 1e-16)
    hh = h.reshape(n, heads, ch)
    return _seg_sum(hh[src] * attn[:, :, None], dst, n)


@jax.jit
def kernel(x, params, edge_index, batch):
    p = params
    f32 = jnp.float32
    grid = (_N // _TILE,)

    loop = jnp.arange(_N, dtype=edge_index.dtype)
    src = jnp.concatenate([edge_index[0], loop])
    dst = jnp.concatenate([edge_index[1], loop])

    a1 = _attn_mat(p['as1'], p['ad1'])
    h1, s1 = pl.pallas_call(
        _k1_body,
        grid=grid,
        in_specs=[_rows(28), _full((28, 256)), _full((256, 8))],
        out_specs=[_rows(256), _rows(8)],
        out_shape=[jax.ShapeDtypeStruct((_N, 256), f32),
                   jax.ShapeDtypeStruct((_N, 8), f32)],
    )(x, p['W1'], a1)

    agg1 = _gat_edges(h1, s1, src, dst, 4, 64).reshape(_N, 256)

    g1, b1, rm1, rv1 = p['bn1']
    hg = pl.pallas_call(
        _k2_body,
        grid=grid,
        in_specs=[_rows(256)] + [_full((1, 256))] * 5 + [_full((256, 64))],
        out_specs=_rows(64),
        out_shape=jax.ShapeDtypeStruct((_N, 64), f32),
    )(agg1, _r2(p['b1']), _r2(g1), _r2(b1), _r2(rm1), _r2(rv1), p['Wg'])

    deg = _seg_sum(jnp.ones(src.shape[0], f32), dst, _N)
    dinv = jnp.where(deg > 0, deg ** -0.5, 0.0)
    aggg = _seg_sum(hg[src] * (dinv[src] * dinv[dst])[:, None], dst, _N)

    g2, b2, rm2, rv2 = p['bn2']
    x3, xr = pl.pallas_call(
        _k3_body,
        grid=grid,
        in_specs=[_rows(64)] + [_full((1, 64))] * 5 + [_full((64, 64))],
        out_specs=[_rows(64), _rows(64)],
        out_shape=[jax.ShapeDtypeStruct((_N, 64), f32),
                   jax.ShapeDtypeStruct((_N, 64), f32)],
    )(aggg, _r2(p['bg']), _r2(g2), _r2(b2), _r2(rm2), _r2(rv2), p['Wsr'])

    es, ed = edge_index[0], edge_index[1]
    cnt = _seg_sum(jnp.ones(es.shape[0], f32), ed, _N)
    aggs = _seg_sum(x3[es], ed, _N) / jnp.maximum(cnt, 1.0)[:, None]

    g3, b3, rm3, rv3 = p['bn3']
    a2 = _attn_mat(p['as2'], p['ad2'])
    h2, s2 = pl.pallas_call(
        _k4_body,
        grid=grid,
        in_specs=[_rows(64), _full((64, 64)), _full((1, 64)), _rows(64)]
                 + [_full((1, 64))] * 4 + [_full((64, 256)), _full((256, 8))],
        out_specs=[_rows(256), _rows(8)],
        out_shape=[jax.ShapeDtypeStruct((_N, 256), f32),
                   jax.ShapeDtypeStruct((_N, 8), f32)],
    )(aggs, p['Wsl'], _r2(p['bsl']), xr, _r2(g3), _r2(b3), _r2(rm3),
      _r2(rv3), p['W2'], a2)

    agg2 = _gat_edges(h2, s2, src, dst, 4, 64).mean(1)

    g4, b4, rm4, rv4 = p['bn4']
    x5 = pl.pallas_call(
        _k5_body,
        grid=grid,
        in_specs=[_rows(64)] + [_full((1, 64))] * 5,
        out_specs=_rows(64),
        out_shape=jax.ShapeDtypeStruct((_N, 64), f32),
    )(agg2, _r2(p['b2']), _r2(g4), _r2(b4), _r2(rm4), _r2(rv4))

    gcnt = _seg_sum(jnp.ones(_N, f32), batch, _G)
    xm = _seg_sum(x5, batch, _G) / jnp.maximum(gcnt, 1.0)[:, None]
    xx = jax.ops.segment_max(x5, batch, num_segments=_G)
    xx = jnp.where(jnp.isfinite(xx), xx, 0.0)
    z0 = jnp.concatenate([xm, xx], axis=1)

    gm1, bm1, rmm1, rvm1 = p['bnm1']
    gm2, bm2, rmm2, rvm2 = p['bnm2']
    out = pl.pallas_call(
        _k6_body,
        grid=(1,),
        in_specs=[_full((_G, 128)),
                  _full((128, 64)), _full((1, 64)), _full((1, 64)),
                  _full((1, 64)), _full((1, 64)), _full((1, 64)),
                  _full((64, 32)), _full((1, 32)), _full((1, 32)),
                  _full((1, 32)), _full((1, 32)), _full((1, 32)),
                  _full((32, 16)), _full((1, 16)),
                  _full((16, 32)), _full((1, 32)),
                  _full((32, 1)), _full((1, 1))],
        out_specs=_full((_G, 1)),
        out_shape=jax.ShapeDtypeStruct((_G, 1), f32),
    )(z0, p['L1W'], _r2(p['L1b']), _r2(gm1), _r2(bm1), _r2(rmm1), _r2(rvm1),
      p['L2W'], _r2(p['L2b']), _r2(gm2), _r2(bm2), _r2(rmm2), _r2(rvm2),
      p['L3W'], _r2(p['L3b']), p['L4W'], _r2(p['L4b']),
      p['L5W'], _r2(p['L5b']))
    return out
